# baseline (speedup 1.0000x reference)
import functools

import jax
import jax.numpy as jnp
from jax import lax
from jax.experimental import pallas as pl
from jax.experimental.pallas import tpu as pltpu
from jax.experimental.pallas import tpu_sc as plsc

_NUM_BLOCKS = 64
_N = 32768
_BT = 4096
_NW = 32
_CHUNK = _N // _NW


def _gate_body(x_ref, w_ref, b_ref, o_ref):
    logits = lax.dot_general(
        x_ref[...], w_ref[...],
        (((1,), (1,)), ((), ())),
        preferred_element_type=jnp.float32,
    ) + b_ref[...]
    m1 = jnp.max(logits, axis=1, keepdims=True)
    c1 = logits == m1
    c1f = c1.astype(jnp.float32)
    masked = jnp.where(c1, -jnp.inf, logits)
    m2 = jnp.max(masked, axis=1, keepdims=True)
    c2 = masked == m2
    fe = lax.broadcasted_iota(jnp.int32, (_NUM_BLOCKS, _NUM_BLOCKS), 0)
    ee = lax.broadcasted_iota(jnp.int32, (_NUM_BLOCKS, _NUM_BLOCKS), 1)
    tri = (fe <= ee).astype(jnp.float32)
    a = c1f + c2.astype(jnp.float32) * (1.0 / 64.0)
    p = lax.dot_general(a, tri, (((1,), (0,)), ((), ())),
                        preferred_element_type=jnp.float32)
    p2 = jnp.floor(p)
    p1 = (p - p2) * 64.0
    n1 = jnp.sum(c1f, axis=1, keepdims=True)
    first1 = c1 & (p2 == 1.0)
    second = (c1 & (p2 == 2.0)) | (c2 & (n1 == 1.0) & (p1 == 1.0))
    idsf = lax.broadcasted_iota(jnp.int32, logits.shape, 1).astype(jnp.float32)
    pkf = jnp.sum(jnp.where(first1, idsf * 64.0, 0.0)
                  + jnp.where(second, idsf, 0.0), axis=1)
    o_ref[...] = pkf.astype(jnp.int32).reshape(1, 1, _BT)


def _tc_pack(x, W, b):
    n = x.shape[0]
    return pl.pallas_call(
        _gate_body,
        grid=(n // _BT,),
        in_specs=[
            pl.BlockSpec((_BT, x.shape[1]), lambda i: (i, 0)),
            pl.BlockSpec((_NUM_BLOCKS, x.shape[1]), lambda i: (0, 0)),
            pl.BlockSpec((1, _NUM_BLOCKS), lambda i: (0, 0)),
        ],
        out_specs=pl.BlockSpec((1, 1, _BT), lambda i: (i, 0, 0)),
        out_shape=jax.ShapeDtypeStruct((n // _BT, 1, _BT), jnp.int32),
    )(x, W, b[None, :])


def _sc_expand_body(pk_hbm, out_hbm, pk_v, mask_v):
    wid = lax.axis_index("s") * 2 + lax.axis_index("c")
    base = wid * _CHUNK
    pltpu.sync_copy(pk_hbm.at[pl.ds(base, _CHUNK)], pk_v)
    zeros16 = jnp.zeros((16,), jnp.float32)
    ones16 = jnp.full((16,), 1.0, jnp.float32)
    iota16 = lax.broadcasted_iota(jnp.int32, (16,), 0)

    def zbody(i, carry):
        for u in range(16):
            mask_v[pl.ds(i * 256 + u * 16, 16)] = zeros16
        return carry

    lax.fori_loop(0, _CHUNK * _NUM_BLOCKS // 256, zbody, 0)

    for j in range(_CHUNK // 16):
        pk = pk_v[pl.ds(j * 16, 16)]
        tok = j * 16 + iota16
        i1 = lax.shift_right_logical(pk, 6)
        i2 = lax.bitwise_and(pk, 63)
        flat = tok * _NUM_BLOCKS
        plsc.store_scatter(mask_v, [flat + i1], ones16)
        plsc.store_scatter(mask_v, [flat + i2], ones16)
    pltpu.sync_copy(mask_v, out_hbm.at[pl.ds(base * _NUM_BLOCKS,
                                             _CHUNK * _NUM_BLOCKS)])


@functools.lru_cache(maxsize=1)
def _sc_expand():
    return pl.kernel(
        _sc_expand_body,
        out_type=jax.ShapeDtypeStruct((_N * _NUM_BLOCKS,), jnp.float32),
        mesh=plsc.VectorSubcoreMesh(core_axis_name="c", subcore_axis_name="s"),
        scratch_types=[
            pltpu.VMEM((_CHUNK,), jnp.int32),
            pltpu.VMEM((_CHUNK * _NUM_BLOCKS,), jnp.float32),
        ],
        compiler_params=pltpu.CompilerParams(needs_layout_passes=False),
    )


def kernel(x, W, b):
    pk = _tc_pack(x, W, b).reshape(_N)
    return _sc_expand()(pk).reshape(_N, _NUM_BLOCKS)


# speedup vs baseline: 1.6769x; 1.6769x over previous
import jax
import jax.numpy as jnp
from jax.experimental import pallas as pl

_NUM_BLOCKS = 64
_BT = 4096
_BH = _BT // 2


def _top2_mask(logits):
    m1 = jnp.max(logits, axis=1, keepdims=True)
    c1 = logits == m1
    c1f = c1.astype(jnp.float32)
    masked = jnp.where(c1, -jnp.inf, logits)
    m2 = jnp.max(masked, axis=1, keepdims=True)
    c2 = masked == m2
    fe = jax.lax.broadcasted_iota(jnp.int32, (_NUM_BLOCKS, _NUM_BLOCKS), 0)
    ee = jax.lax.broadcasted_iota(jnp.int32, (_NUM_BLOCKS, _NUM_BLOCKS), 1)
    tri = (fe <= ee).astype(jnp.float32)
    a = c1f + c2.astype(jnp.float32) * (1.0 / 64.0)
    p = jax.lax.dot_general(a, tri, (((1,), (0,)), ((), ())),
                            preferred_element_type=jnp.float32)
    p2 = jnp.floor(p)
    p1 = (p - p2) * 64.0
    n1 = jnp.sum(c1f, axis=1, keepdims=True)
    sel = (c1 & (p2 <= 2.0)) | (c2 & (n1 == 1.0) & (p1 <= 1.0))
    return sel.astype(jnp.float32)


def _gate_body(xa_ref, xb_ref, w_ref, b_ref, o_ref):
    for half, x_ref in ((0, xa_ref), (1, xb_ref)):
        logits = jax.lax.dot_general(
            x_ref[...], w_ref[...],
            (((1,), (1,)), ((), ())),
            preferred_element_type=jnp.float32,
        ) + b_ref[...]
        o_ref[pl.ds(half * _BH, _BH), :] = _top2_mask(logits)


def kernel(x, W, b):
    n = x.shape[0]
    return pl.pallas_call(
        _gate_body,
        grid=(n // _BT,),
        in_specs=[
            pl.BlockSpec((_BH, x.shape[1]), lambda i: (2 * i, 0)),
            pl.BlockSpec((_BH, x.shape[1]), lambda i: (2 * i + 1, 0)),
            pl.BlockSpec((_NUM_BLOCKS, x.shape[1]), lambda i: (0, 0)),
            pl.BlockSpec((1, _NUM_BLOCKS), lambda i: (0, 0)),
        ],
        out_specs=pl.BlockSpec((_BT, _NUM_BLOCKS), lambda i: (i, 0)),
        out_shape=jax.ShapeDtypeStruct((n, _NUM_BLOCKS), jnp.float32),
    )(x, x, W, b[None, :])
